# baseline (speedup 1.0000x reference)
import functools

import jax
import jax.numpy as jnp
from jax import lax
from jax.experimental import pallas as pl
from jax.experimental.pallas import tpu as pltpu
from jax.experimental.pallas import tpu_sc as plsc

E = 8
K = 2
BLK = 128
NBP = 40
CH = 128

_SC_INFO = plsc.get_sparse_core_info()
_NC, _NS, _L = _SC_INFO.num_cores, _SC_INFO.num_subcores, _SC_INFO.num_lanes
_NW = _NC * _NS


def _route_body(x_ref, rwt_ref, slot_ref, p_ref, bexp_ref, oh_scr, c_scr):
    T = x_ref.shape[0]
    logits = jnp.dot(x_ref[...], rwt_ref[...], preferred_element_type=jnp.float32)
    iota_e = lax.broadcasted_iota(jnp.int32, (T, E), 1)
    m1 = jnp.max(logits, axis=1, keepdims=True)
    i1 = jnp.min(jnp.where(logits == m1, iota_e, E), axis=1, keepdims=True)
    l2 = jnp.where(iota_e == i1, -jnp.inf, logits)
    m2 = jnp.max(l2, axis=1, keepdims=True)
    i2 = jnp.min(jnp.where(l2 == m2, iota_e, E), axis=1, keepdims=True)
    p1 = 1.0 / (1.0 + jnp.exp(m2 - m1))
    p2 = 1.0 - p1
    oh1 = (iota_e == i1).astype(jnp.float32)
    oh2 = (iota_e == i2).astype(jnp.float32)
    oh_scr[...] = oh1 + oh2

    lr = lax.broadcasted_iota(jnp.int32, (CH, CH), 0)
    lc = lax.broadcasted_iota(jnp.int32, (CH, CH), 1)
    lstrict = (lr > lc).astype(jnp.float32)

    def step(c, carry):
        sl = pl.ds(c * CH, CH)
        ohc = oh_scr[sl, :]
        c_scr[sl, :] = jnp.dot(lstrict, ohc, preferred_element_type=jnp.float32) + carry
        return carry + jnp.sum(ohc, axis=0, keepdims=True)

    counts = lax.fori_loop(0, T // CH, step, jnp.zeros((1, E), jnp.float32))

    blocks = jnp.floor((counts + (BLK - 1)) * (1.0 / BLK))
    iota_r8 = lax.broadcasted_iota(jnp.int32, (E, E), 0)
    iota_c8 = lax.broadcasted_iota(jnp.int32, (E, E), 1)
    blocks_b = jnp.dot(jnp.ones((E, 1), jnp.float32), blocks,
                       preferred_element_type=jnp.float32)
    blocks_s = jnp.sum(jnp.where(iota_r8 == iota_c8, blocks_b, 0.0),
                       axis=1, keepdims=True)
    lincl = (iota_r8 >= iota_c8).astype(jnp.float32)
    cumb_incl_s = jnp.dot(lincl, blocks_s, preferred_element_type=jnp.float32)
    iota_b = lax.broadcasted_iota(jnp.int32, (E, 128), 1)
    ge = (iota_b >= cumb_incl_s.astype(jnp.int32)).astype(jnp.int32)
    bexp_ref[...] = jnp.minimum(jnp.sum(ge, axis=0, keepdims=True), E - 1)

    cumb_excl = jnp.dot(blocks, (iota_r8 < iota_c8).astype(jnp.float32),
                        preferred_element_type=jnp.float32)
    off_row = cumb_excl * float(BLK)
    c_full = c_scr[...]
    slot1 = jnp.sum(oh1 * (c_full + off_row), axis=1, keepdims=True)
    slot2 = jnp.sum(oh2 * (c_full + off_row), axis=1, keepdims=True)
    slot_ref[...] = jnp.concatenate(
        [slot1.astype(jnp.int32), slot2.astype(jnp.int32)], axis=1)
    iota_l = lax.broadcasted_iota(jnp.int32, (x_ref.shape[0], K * 128), 1)
    p_ref[...] = jnp.where(iota_l < 128, p1, p2)


def _route(xf, rwt):
    T, Dd = xf.shape
    return pl.pallas_call(
        _route_body,
        out_shape=(
            jax.ShapeDtypeStruct((T, K), jnp.int32),
            jax.ShapeDtypeStruct((T, K * 128), jnp.float32),
            jax.ShapeDtypeStruct((1, 128), jnp.int32),
        ),
        scratch_shapes=[pltpu.VMEM((T, E), jnp.float32),
                        pltpu.VMEM((T, E), jnp.float32)],
    )(xf, rwt)


def _make_dispatch(T, Dd, ns_rows):
    npw = (T * K) // _NW
    mesh = plsc.VectorSubcoreMesh(core_axis_name="c", subcore_axis_name="s")

    @functools.partial(
        pl.kernel, mesh=mesh,
        out_type=(
            jax.ShapeDtypeStruct((ns_rows, Dd), jnp.float32),
            jax.ShapeDtypeStruct((ns_rows, 128), jnp.float32),
        ),
        scratch_types=[
            pltpu.VMEM((npw,), jnp.int32),
            pltpu.VMEM((npw,), jnp.int32),
            pltpu.VMEM((npw, Dd), jnp.float32),
            pltpu.VMEM((npw, 128), jnp.float32),
            pltpu.SemaphoreType.DMA,
            pltpu.SemaphoreType.DMA,
            pltpu.SemaphoreType.DMA,
        ],
    )
    def dispatch(xf_hbm, tok_hbm, slot_hbm, p_hbm, xs_hbm, ps_hbm,
                 tok_v, slot_v, xbuf, prep, sem_g, sem_x, sem_p):
        wid = lax.axis_index("s") * _NC + lax.axis_index("c")
        base = wid * npw
        pltpu.sync_copy(tok_hbm.at[pl.ds(base, npw)], tok_v)
        pltpu.sync_copy(slot_hbm.at[pl.ds(base, npw)], slot_v)
        pltpu.sync_copy(p_hbm.at[pl.ds(base, npw)], prep)
        gather = pltpu.async_copy(xf_hbm.at[tok_v], xbuf, sem_g)
        gather.wait()
        cx = pltpu.async_copy(xbuf, xs_hbm.at[slot_v], sem_x)
        cp = pltpu.async_copy(prep, ps_hbm.at[slot_v], sem_p)
        cx.wait()
        cp.wait()

    return dispatch


def _gelu_exact(v):
    return 0.5 * v * (1.0 + lax.erf(v * 0.7071067811865476))


def _mlp_body(bexp_ref, xs_ref, ps_ref, w1_ref, w2_ref, y_ref):
    h = jnp.dot(xs_ref[...], w1_ref[...], preferred_element_type=jnp.float32)
    h = _gelu_exact(h)
    y = jnp.dot(h, w2_ref[...], preferred_element_type=jnp.float32)
    y_ref[...] = y * ps_ref[:, 0:1]


def _grouped_mlp(bexp, xs, ps, w1, w2):
    ns_rows, Dd = xs.shape
    S = w1.shape[1] // E
    grid_spec = pltpu.PrefetchScalarGridSpec(
        num_scalar_prefetch=1,
        grid=(NBP,),
        in_specs=[
            pl.BlockSpec((BLK, Dd), lambda b, be: (b, 0)),
            pl.BlockSpec((BLK, 128), lambda b, be: (b, 0)),
            pl.BlockSpec((Dd, S), lambda b, be: (0, be[b])),
            pl.BlockSpec((S, Dd), lambda b, be: (be[b], 0)),
        ],
        out_specs=pl.BlockSpec((BLK, Dd), lambda b, be: (b, 0)),
    )
    return pl.pallas_call(
        _mlp_body,
        grid_spec=grid_spec,
        out_shape=jax.ShapeDtypeStruct((ns_rows, Dd), jnp.float32),
    )(bexp, xs, ps, w1, w2)


def _make_combine(T, Dd, ns_rows):
    npw = (T * K) // _NW
    tpw = T // _NW
    nch = Dd // _L
    mesh = plsc.VectorSubcoreMesh(core_axis_name="c", subcore_axis_name="s")

    hpw = npw // 2
    htw = tpw // 2

    @functools.partial(
        pl.kernel, mesh=mesh,
        out_type=jax.ShapeDtypeStruct((T, Dd), jnp.float32),
        scratch_types=[
            pltpu.VMEM((hpw,), jnp.int32),
            pltpu.VMEM((hpw, Dd), jnp.float32),
            pltpu.VMEM((htw, Dd), jnp.float32),
            pltpu.SemaphoreType.DMA,
        ],
    )
    def combine(y_hbm, slot_hbm, out_hbm, slot_v, ybuf, obuf, sem):
        wid = lax.axis_index("s") * _NC + lax.axis_index("c")
        for half in range(2):
            pltpu.sync_copy(slot_hbm.at[pl.ds(wid * npw + half * hpw, hpw)], slot_v)
            pltpu.async_copy(y_hbm.at[slot_v], ybuf, sem).wait()

            def tok(i, _):
                for c in range(nch):
                    sl = pl.ds(c * _L, _L)
                    obuf[i, sl] = ybuf[2 * i, sl] + ybuf[2 * i + 1, sl]
                return 0

            lax.fori_loop(0, htw, tok, 0)
            pltpu.sync_copy(obuf, out_hbm.at[pl.ds(wid * tpw + half * htw, htw)])

    return combine


def kernel(x, w1, w2, router_w):
    Bb, Ss, Dd = x.shape
    T = Bb * Ss
    ns_rows = NBP * BLK
    xf = x.reshape(T, Dd)
    rwt = router_w.T

    slot, p, bexp = _route(xf, rwt)
    slot_flat = slot.reshape(T * K)
    p_rep = p.reshape(T * K, 128)
    bexp_flat = bexp.reshape(128)[:NBP]
    tok_ids = jnp.repeat(jnp.arange(T, dtype=jnp.int32), K)

    xs, ps = _make_dispatch(T, Dd, ns_rows)(xf, tok_ids, slot_flat, p_rep)
    y = _grouped_mlp(bexp_flat, xs, ps, w1, w2)
    out = _make_combine(T, Dd, ns_rows)(y, slot_flat)
    return out.reshape(Bb, Ss, Dd)


# speedup vs baseline: 1.0036x; 1.0036x over previous
import functools

import jax
import jax.numpy as jnp
from jax import lax
from jax.experimental import pallas as pl
from jax.experimental.pallas import tpu as pltpu
from jax.experimental.pallas import tpu_sc as plsc

E = 8
K = 2
BLK = 256
NBP = 24
CH = 256

_SC_INFO = plsc.get_sparse_core_info()
_NC, _NS, _L = _SC_INFO.num_cores, _SC_INFO.num_subcores, _SC_INFO.num_lanes
_NW = _NC * _NS


def _route_body(x_ref, rwt_ref, slot_ref, p_ref, bexp_ref, oh_scr, c_scr):
    T = x_ref.shape[0]
    logits = jnp.dot(x_ref[...], rwt_ref[...], preferred_element_type=jnp.float32)
    iota_e = lax.broadcasted_iota(jnp.int32, (T, E), 1)
    m1 = jnp.max(logits, axis=1, keepdims=True)
    i1 = jnp.min(jnp.where(logits == m1, iota_e, E), axis=1, keepdims=True)
    l2 = jnp.where(iota_e == i1, -jnp.inf, logits)
    m2 = jnp.max(l2, axis=1, keepdims=True)
    i2 = jnp.min(jnp.where(l2 == m2, iota_e, E), axis=1, keepdims=True)
    p1 = 1.0 / (1.0 + jnp.exp(m2 - m1))
    p2 = 1.0 - p1
    oh1 = (iota_e == i1).astype(jnp.float32)
    oh2 = (iota_e == i2).astype(jnp.float32)
    oh_scr[...] = oh1 + oh2

    lr = lax.broadcasted_iota(jnp.int32, (CH, CH), 0)
    lc = lax.broadcasted_iota(jnp.int32, (CH, CH), 1)
    lstrict = (lr > lc).astype(jnp.float32)

    def step(c, carry):
        sl = pl.ds(c * CH, CH)
        ohc = oh_scr[sl, :]
        c_scr[sl, :] = jnp.dot(lstrict, ohc, preferred_element_type=jnp.float32) + carry
        return carry + jnp.sum(ohc, axis=0, keepdims=True)

    counts = lax.fori_loop(0, T // CH, step, jnp.zeros((1, E), jnp.float32))

    blocks = jnp.floor((counts + (BLK - 1)) * (1.0 / BLK))
    iota_r8 = lax.broadcasted_iota(jnp.int32, (E, E), 0)
    iota_c8 = lax.broadcasted_iota(jnp.int32, (E, E), 1)
    blocks_b = jnp.dot(jnp.ones((E, 1), jnp.float32), blocks,
                       preferred_element_type=jnp.float32)
    blocks_s = jnp.sum(jnp.where(iota_r8 == iota_c8, blocks_b, 0.0),
                       axis=1, keepdims=True)
    lincl = (iota_r8 >= iota_c8).astype(jnp.float32)
    cumb_incl_s = jnp.dot(lincl, blocks_s, preferred_element_type=jnp.float32)
    iota_b = lax.broadcasted_iota(jnp.int32, (E, 128), 1)
    ge = (iota_b >= cumb_incl_s.astype(jnp.int32)).astype(jnp.int32)
    bexp_ref[...] = jnp.minimum(jnp.sum(ge, axis=0, keepdims=True), E - 1)

    cumb_excl = jnp.dot(blocks, (iota_r8 < iota_c8).astype(jnp.float32),
                        preferred_element_type=jnp.float32)
    off_row = cumb_excl * float(BLK)
    c_full = c_scr[...]
    slot1 = jnp.sum(oh1 * (c_full + off_row), axis=1, keepdims=True)
    slot2 = jnp.sum(oh2 * (c_full + off_row), axis=1, keepdims=True)
    slot_ref[...] = jnp.concatenate(
        [slot1.astype(jnp.int32), slot2.astype(jnp.int32)], axis=1)
    iota_l = lax.broadcasted_iota(jnp.int32, (x_ref.shape[0], K * 128), 1)
    p_ref[...] = jnp.where(iota_l < 128, p1, p2)


def _route(xf, rwt):
    T, Dd = xf.shape
    return pl.pallas_call(
        _route_body,
        out_shape=(
            jax.ShapeDtypeStruct((T, K), jnp.int32),
            jax.ShapeDtypeStruct((T, K * 128), jnp.float32),
            jax.ShapeDtypeStruct((1, 128), jnp.int32),
        ),
        scratch_shapes=[pltpu.VMEM((T, E), jnp.float32),
                        pltpu.VMEM((T, E), jnp.float32)],
    )(xf, rwt)


def _make_dispatch(T, Dd, ns_rows):
    npw = (T * K) // _NW
    mesh = plsc.VectorSubcoreMesh(core_axis_name="c", subcore_axis_name="s")

    @functools.partial(
        pl.kernel, mesh=mesh,
        out_type=(
            jax.ShapeDtypeStruct((ns_rows, Dd), jnp.float32),
            jax.ShapeDtypeStruct((ns_rows, 128), jnp.float32),
        ),
        scratch_types=[
            pltpu.VMEM((npw,), jnp.int32),
            pltpu.VMEM((npw,), jnp.int32),
            pltpu.VMEM((npw, Dd), jnp.float32),
            pltpu.VMEM((npw, 128), jnp.float32),
            pltpu.SemaphoreType.DMA,
            pltpu.SemaphoreType.DMA,
            pltpu.SemaphoreType.DMA,
        ],
    )
    def dispatch(xf_hbm, tok_hbm, slot_hbm, p_hbm, xs_hbm, ps_hbm,
                 tok_v, slot_v, xbuf, prep, sem_g, sem_x, sem_p):
        wid = lax.axis_index("s") * _NC + lax.axis_index("c")
        base = wid * npw
        pltpu.sync_copy(tok_hbm.at[pl.ds(base, npw)], tok_v)
        pltpu.sync_copy(slot_hbm.at[pl.ds(base, npw)], slot_v)
        pltpu.sync_copy(p_hbm.at[pl.ds(base, npw)], prep)
        gather = pltpu.async_copy(xf_hbm.at[tok_v], xbuf, sem_g)
        gather.wait()
        cx = pltpu.async_copy(xbuf, xs_hbm.at[slot_v], sem_x)
        cp = pltpu.async_copy(prep, ps_hbm.at[slot_v], sem_p)
        cx.wait()
        cp.wait()

    return dispatch


def _gelu_exact(v):
    return 0.5 * v * (1.0 + lax.erf(v * 0.7071067811865476))


def _mlp_body(bexp_ref, xs_ref, ps_ref, w1_ref, w2_ref, y_ref):
    h = jnp.dot(xs_ref[...], w1_ref[...], preferred_element_type=jnp.float32)
    h = _gelu_exact(h)
    y = jnp.dot(h, w2_ref[...], preferred_element_type=jnp.float32)
    y_ref[...] = y * ps_ref[:, 0:1]


def _grouped_mlp(bexp, xs, ps, w1, w2):
    ns_rows, Dd = xs.shape
    S = w1.shape[1] // E
    grid_spec = pltpu.PrefetchScalarGridSpec(
        num_scalar_prefetch=1,
        grid=(NBP,),
        in_specs=[
            pl.BlockSpec((BLK, Dd), lambda b, be: (b, 0)),
            pl.BlockSpec((BLK, 128), lambda b, be: (b, 0)),
            pl.BlockSpec((Dd, S), lambda b, be: (0, be[b])),
            pl.BlockSpec((S, Dd), lambda b, be: (be[b], 0)),
        ],
        out_specs=pl.BlockSpec((BLK, Dd), lambda b, be: (b, 0)),
    )
    return pl.pallas_call(
        _mlp_body,
        grid_spec=grid_spec,
        out_shape=jax.ShapeDtypeStruct((ns_rows, Dd), jnp.float32),
    )(bexp, xs, ps, w1, w2)


def _make_combine(T, Dd, ns_rows):
    npw = (T * K) // _NW
    tpw = T // _NW
    nch = Dd // _L
    mesh = plsc.VectorSubcoreMesh(core_axis_name="c", subcore_axis_name="s")

    hpw = npw // 2
    htw = tpw // 2

    @functools.partial(
        pl.kernel, mesh=mesh,
        out_type=jax.ShapeDtypeStruct((T, Dd), jnp.float32),
        scratch_types=[
            pltpu.VMEM((hpw,), jnp.int32),
            pltpu.VMEM((hpw, Dd), jnp.float32),
            pltpu.VMEM((htw, Dd), jnp.float32),
            pltpu.SemaphoreType.DMA,
        ],
    )
    def combine(y_hbm, slot_hbm, out_hbm, slot_v, ybuf, obuf, sem):
        wid = lax.axis_index("s") * _NC + lax.axis_index("c")
        for half in range(2):
            pltpu.sync_copy(slot_hbm.at[pl.ds(wid * npw + half * hpw, hpw)], slot_v)
            pltpu.async_copy(y_hbm.at[slot_v], ybuf, sem).wait()

            def grp(g, _):
                ob = obuf.at[pl.ds(g * 8, 8)]
                yb = ybuf.at[pl.ds(g * 16, 16)]
                for ii in range(8):
                    for c in range(nch):
                        sl = pl.ds(c * _L, _L)
                        ob[ii, sl] = yb[2 * ii, sl] + yb[2 * ii + 1, sl]
                return 0

            lax.fori_loop(0, htw // 8, grp, 0)
            pltpu.sync_copy(obuf, out_hbm.at[pl.ds(wid * tpw + half * htw, htw)])

    return combine


def kernel(x, w1, w2, router_w):
    Bb, Ss, Dd = x.shape
    T = Bb * Ss
    ns_rows = NBP * BLK
    xf = x.reshape(T, Dd)
    rwt = router_w.T

    slot, p, bexp = _route(xf, rwt)
    slot_flat = slot.reshape(T * K)
    p_rep = p.reshape(T * K, 128)
    bexp_flat = bexp.reshape(128)[:NBP]
    tok_ids = jnp.repeat(jnp.arange(T, dtype=jnp.int32), K)

    xs, ps = _make_dispatch(T, Dd, ns_rows)(xf, tok_ids, slot_flat, p_rep)
    y = _grouped_mlp(bexp_flat, xs, ps, w1, w2)
    out = _make_combine(T, Dd, ns_rows)(y, slot_flat)
    return out.reshape(Bb, Ss, Dd)


# speedup vs baseline: 1.1306x; 1.1266x over previous
import functools

import jax
import jax.numpy as jnp
from jax import lax
from jax.experimental import pallas as pl
from jax.experimental.pallas import tpu as pltpu
from jax.experimental.pallas import tpu_sc as plsc

E = 8
K = 2
BLK = 256
NBP = 24
CH = 256

_SC_INFO = plsc.get_sparse_core_info()
_NC, _NS, _L = _SC_INFO.num_cores, _SC_INFO.num_subcores, _SC_INFO.num_lanes
_NW = _NC * _NS


def _route_body(x_ref, rwt_ref, slot_ref, p_ref, bexp_ref, oh_scr, c_scr):
    T = x_ref.shape[0]
    logits = jnp.dot(x_ref[...], rwt_ref[...], preferred_element_type=jnp.float32)
    iota_e = lax.broadcasted_iota(jnp.int32, (T, E), 1)
    m1 = jnp.max(logits, axis=1, keepdims=True)
    i1 = jnp.min(jnp.where(logits == m1, iota_e, E), axis=1, keepdims=True)
    l2 = jnp.where(iota_e == i1, -jnp.inf, logits)
    m2 = jnp.max(l2, axis=1, keepdims=True)
    i2 = jnp.min(jnp.where(l2 == m2, iota_e, E), axis=1, keepdims=True)
    p1 = 1.0 / (1.0 + jnp.exp(m2 - m1))
    p2 = 1.0 - p1
    oh1 = (iota_e == i1).astype(jnp.float32)
    oh2 = (iota_e == i2).astype(jnp.float32)
    oh_scr[...] = oh1 + oh2

    lr = lax.broadcasted_iota(jnp.int32, (CH, CH), 0)
    lc = lax.broadcasted_iota(jnp.int32, (CH, CH), 1)
    lstrict = (lr > lc).astype(jnp.float32)

    def step(c, carry):
        sl = pl.ds(c * CH, CH)
        ohc = oh_scr[sl, :]
        c_scr[sl, :] = jnp.dot(lstrict, ohc, preferred_element_type=jnp.float32) + carry
        return carry + jnp.sum(ohc, axis=0, keepdims=True)

    counts = lax.fori_loop(0, T // CH, step, jnp.zeros((1, E), jnp.float32))

    blocks = jnp.floor((counts + (BLK - 1)) * (1.0 / BLK))
    iota_r8 = lax.broadcasted_iota(jnp.int32, (E, E), 0)
    iota_c8 = lax.broadcasted_iota(jnp.int32, (E, E), 1)
    blocks_b = jnp.dot(jnp.ones((E, 1), jnp.float32), blocks,
                       preferred_element_type=jnp.float32)
    blocks_s = jnp.sum(jnp.where(iota_r8 == iota_c8, blocks_b, 0.0),
                       axis=1, keepdims=True)
    lincl = (iota_r8 >= iota_c8).astype(jnp.float32)
    cumb_incl_s = jnp.dot(lincl, blocks_s, preferred_element_type=jnp.float32)
    iota_b = lax.broadcasted_iota(jnp.int32, (E, 128), 1)
    ge = (iota_b >= cumb_incl_s.astype(jnp.int32)).astype(jnp.int32)
    bexp_ref[...] = jnp.minimum(jnp.sum(ge, axis=0, keepdims=True), E - 1)

    cumb_excl = jnp.dot(blocks, (iota_r8 < iota_c8).astype(jnp.float32),
                        preferred_element_type=jnp.float32)
    off_row = cumb_excl * float(BLK)
    c_full = c_scr[...]
    slot1 = jnp.sum(oh1 * (c_full + off_row), axis=1, keepdims=True)
    slot2 = jnp.sum(oh2 * (c_full + off_row), axis=1, keepdims=True)
    slot_ref[...] = jnp.concatenate(
        [slot1.astype(jnp.int32), slot2.astype(jnp.int32)], axis=1)
    iota_l = lax.broadcasted_iota(jnp.int32, (x_ref.shape[0], K * 128), 1)
    p_ref[...] = jnp.where(iota_l < 128, p1, p2)


def _route(xf, rwt):
    T, Dd = xf.shape
    return pl.pallas_call(
        _route_body,
        out_shape=(
            jax.ShapeDtypeStruct((T, K), jnp.int32),
            jax.ShapeDtypeStruct((T, K * 128), jnp.float32),
            jax.ShapeDtypeStruct((1, 128), jnp.int32),
        ),
        scratch_shapes=[pltpu.VMEM((T, E), jnp.float32),
                        pltpu.VMEM((T, E), jnp.float32)],
    )(xf, rwt)


def _make_dispatch(T, Dd, ns_rows):
    npw = (T * K) // _NW
    mesh = plsc.VectorSubcoreMesh(core_axis_name="c", subcore_axis_name="s")

    @functools.partial(
        pl.kernel, mesh=mesh,
        out_type=(
            jax.ShapeDtypeStruct((ns_rows, Dd), jnp.float32),
            jax.ShapeDtypeStruct((ns_rows, 128), jnp.float32),
        ),
        scratch_types=[
            pltpu.VMEM((npw,), jnp.int32),
            pltpu.VMEM((npw,), jnp.int32),
            pltpu.VMEM((npw, Dd), jnp.float32),
            pltpu.VMEM((npw, 128), jnp.float32),
            pltpu.SemaphoreType.DMA,
            pltpu.SemaphoreType.DMA,
            pltpu.SemaphoreType.DMA,
        ],
    )
    def dispatch(xf_hbm, tok_hbm, slot_hbm, p_hbm, xs_hbm, ps_hbm,
                 tok_v, slot_v, xbuf, prep, sem_g, sem_x, sem_p):
        wid = lax.axis_index("s") * _NC + lax.axis_index("c")
        base = wid * npw
        pltpu.sync_copy(tok_hbm.at[pl.ds(base, npw)], tok_v)
        pltpu.sync_copy(slot_hbm.at[pl.ds(base, npw)], slot_v)
        pltpu.sync_copy(p_hbm.at[pl.ds(base, npw)], prep)
        gather = pltpu.async_copy(xf_hbm.at[tok_v], xbuf, sem_g)
        gather.wait()
        cx = pltpu.async_copy(xbuf, xs_hbm.at[slot_v], sem_x)
        cp = pltpu.async_copy(prep, ps_hbm.at[slot_v], sem_p)
        cx.wait()
        cp.wait()

    return dispatch


def _gelu_exact(v):
    return 0.5 * v * (1.0 + lax.erf(v * 0.7071067811865476))


def _mlp_body(bexp_ref, xs_ref, ps_ref, w1_ref, w2_ref, y_ref):
    h = jnp.dot(xs_ref[...].astype(jnp.bfloat16), w1_ref[...].astype(jnp.bfloat16),
                preferred_element_type=jnp.float32)
    h = _gelu_exact(h)
    y = jnp.dot(h.astype(jnp.bfloat16), w2_ref[...].astype(jnp.bfloat16),
                preferred_element_type=jnp.float32)
    y_ref[...] = y * ps_ref[:, 0:1]


def _grouped_mlp(bexp, xs, ps, w1, w2):
    ns_rows, Dd = xs.shape
    S = w1.shape[1] // E
    grid_spec = pltpu.PrefetchScalarGridSpec(
        num_scalar_prefetch=1,
        grid=(NBP,),
        in_specs=[
            pl.BlockSpec((BLK, Dd), lambda b, be: (b, 0)),
            pl.BlockSpec((BLK, 128), lambda b, be: (b, 0)),
            pl.BlockSpec((Dd, S), lambda b, be: (0, be[b])),
            pl.BlockSpec((S, Dd), lambda b, be: (be[b], 0)),
        ],
        out_specs=pl.BlockSpec((BLK, Dd), lambda b, be: (b, 0)),
    )
    return pl.pallas_call(
        _mlp_body,
        grid_spec=grid_spec,
        out_shape=jax.ShapeDtypeStruct((ns_rows, Dd), jnp.float32),
    )(bexp, xs, ps, w1, w2)


def _make_combine(T, Dd, ns_rows):
    npw = (T * K) // _NW
    tpw = T // _NW
    nch = Dd // _L
    mesh = plsc.VectorSubcoreMesh(core_axis_name="c", subcore_axis_name="s")

    @functools.partial(
        pl.kernel, mesh=mesh,
        out_type=jax.ShapeDtypeStruct((T * K, Dd), jnp.float32),
        scratch_types=[
            pltpu.VMEM((npw,), jnp.int32),
            pltpu.VMEM((npw, Dd), jnp.float32),
            pltpu.SemaphoreType.DMA,
        ],
    )
    def combine_gather(y_hbm, slot_hbm, yg_hbm, slot_v, ybuf, sem):
        wid = lax.axis_index("s") * _NC + lax.axis_index("c")
        base = wid * npw
        pltpu.sync_copy(slot_hbm.at[pl.ds(base, npw)], slot_v)
        pltpu.async_copy(y_hbm.at[slot_v], ybuf, sem).wait()
        pltpu.sync_copy(ybuf, yg_hbm.at[pl.ds(base, npw)])

    return combine_gather


def _pair_add_body(yg_ref, o_ref):
    R = o_ref.shape[0]
    ir = lax.broadcasted_iota(jnp.int32, (R, 2 * R), 0)
    ic = lax.broadcasted_iota(jnp.int32, (R, 2 * R), 1)
    pmat = ((ic == 2 * ir) | (ic == 2 * ir + 1)).astype(jnp.float32)
    o_ref[...] = jnp.dot(pmat, yg_ref[...], preferred_element_type=jnp.float32)


def _pair_add(yg, T, Dd):
    R = 256
    return pl.pallas_call(
        _pair_add_body,
        grid=(T // R,),
        in_specs=[pl.BlockSpec((2 * R, Dd), lambda w: (w, 0))],
        out_specs=pl.BlockSpec((R, Dd), lambda w: (w, 0)),
        out_shape=jax.ShapeDtypeStruct((T, Dd), jnp.float32),
    )(yg)


def kernel(x, w1, w2, router_w):
    Bb, Ss, Dd = x.shape
    T = Bb * Ss
    ns_rows = NBP * BLK
    xf = x.reshape(T, Dd)
    rwt = router_w.T

    slot, p, bexp = _route(xf, rwt)
    slot_flat = slot.reshape(T * K)
    p_rep = p.reshape(T * K, 128)
    bexp_flat = bexp.reshape(128)[:NBP]
    tok_ids = jnp.repeat(jnp.arange(T, dtype=jnp.int32), K)

    xs, ps = _make_dispatch(T, Dd, ns_rows)(xf, tok_ids, slot_flat, p_rep)
    y = _grouped_mlp(bexp_flat, xs, ps, w1, w2)
    yg = _make_combine(T, Dd, ns_rows)(y, slot_flat)
    out = _pair_add(yg, T, Dd)
    return out.reshape(Bb, Ss, Dd)


# speedup vs baseline: 1.1321x; 1.0013x over previous
import functools

import jax
import jax.numpy as jnp
from jax import lax
from jax.experimental import pallas as pl
from jax.experimental.pallas import tpu as pltpu
from jax.experimental.pallas import tpu_sc as plsc

E = 8
K = 2
BLK = 256
NBP = 24
CH = 512

_SC_INFO = plsc.get_sparse_core_info()
_NC, _NS, _L = _SC_INFO.num_cores, _SC_INFO.num_subcores, _SC_INFO.num_lanes
_NW = _NC * _NS


def _route_body(x_ref, rwt_ref, slot_ref, p_ref, bexp_ref, oh_scr, c_scr):
    T = x_ref.shape[0]
    logits = jnp.dot(x_ref[...], rwt_ref[...], preferred_element_type=jnp.float32)
    iota_e = lax.broadcasted_iota(jnp.int32, (T, E), 1)
    m1 = jnp.max(logits, axis=1, keepdims=True)
    i1 = jnp.min(jnp.where(logits == m1, iota_e, E), axis=1, keepdims=True)
    l2 = jnp.where(iota_e == i1, -jnp.inf, logits)
    m2 = jnp.max(l2, axis=1, keepdims=True)
    i2 = jnp.min(jnp.where(l2 == m2, iota_e, E), axis=1, keepdims=True)
    p1 = 1.0 / (1.0 + jnp.exp(m2 - m1))
    p2 = 1.0 - p1
    oh1 = (iota_e == i1).astype(jnp.float32)
    oh2 = (iota_e == i2).astype(jnp.float32)
    oh_scr[...] = oh1 + oh2

    lr = lax.broadcasted_iota(jnp.int32, (CH, CH), 0)
    lc = lax.broadcasted_iota(jnp.int32, (CH, CH), 1)
    lstrict = (lr > lc).astype(jnp.float32)

    def step(c, carry):
        sl = pl.ds(c * CH, CH)
        ohc = oh_scr[sl, :]
        c_scr[sl, :] = jnp.dot(lstrict, ohc, preferred_element_type=jnp.float32) + carry
        return carry + jnp.sum(ohc, axis=0, keepdims=True)

    counts = lax.fori_loop(0, T // CH, step, jnp.zeros((1, E), jnp.float32))

    blocks = jnp.floor((counts + (BLK - 1)) * (1.0 / BLK))
    iota_r8 = lax.broadcasted_iota(jnp.int32, (E, E), 0)
    iota_c8 = lax.broadcasted_iota(jnp.int32, (E, E), 1)
    blocks_b = jnp.dot(jnp.ones((E, 1), jnp.float32), blocks,
                       preferred_element_type=jnp.float32)
    blocks_s = jnp.sum(jnp.where(iota_r8 == iota_c8, blocks_b, 0.0),
                       axis=1, keepdims=True)
    lincl = (iota_r8 >= iota_c8).astype(jnp.float32)
    cumb_incl_s = jnp.dot(lincl, blocks_s, preferred_element_type=jnp.float32)
    iota_b = lax.broadcasted_iota(jnp.int32, (E, 128), 1)
    ge = (iota_b >= cumb_incl_s.astype(jnp.int32)).astype(jnp.int32)
    bexp_ref[...] = jnp.minimum(jnp.sum(ge, axis=0, keepdims=True), E - 1)

    cumb_excl = jnp.dot(blocks, (iota_r8 < iota_c8).astype(jnp.float32),
                        preferred_element_type=jnp.float32)
    off_row = cumb_excl * float(BLK)
    c_full = c_scr[...]
    slot1 = jnp.sum(oh1 * (c_full + off_row), axis=1, keepdims=True)
    slot2 = jnp.sum(oh2 * (c_full + off_row), axis=1, keepdims=True)
    slot_ref[...] = jnp.concatenate(
        [slot1.astype(jnp.int32), slot2.astype(jnp.int32)], axis=1)
    iota_l = lax.broadcasted_iota(jnp.int32, (x_ref.shape[0], K * 128), 1)
    p_ref[...] = jnp.where(iota_l < 128, p1, p2)


def _route(xf, rwt):
    T, Dd = xf.shape
    return pl.pallas_call(
        _route_body,
        out_shape=(
            jax.ShapeDtypeStruct((T, K), jnp.int32),
            jax.ShapeDtypeStruct((T, K * 128), jnp.float32),
            jax.ShapeDtypeStruct((1, 128), jnp.int32),
        ),
        scratch_shapes=[pltpu.VMEM((T, E), jnp.float32),
                        pltpu.VMEM((T, E), jnp.float32)],
    )(xf, rwt)


def _make_dispatch(T, Dd, ns_rows):
    npw = (T * K) // _NW
    hpw = npw // 2
    mesh = plsc.VectorSubcoreMesh(core_axis_name="c", subcore_axis_name="s")

    @functools.partial(
        pl.kernel, mesh=mesh,
        out_type=(
            jax.ShapeDtypeStruct((ns_rows, Dd), jnp.float32),
            jax.ShapeDtypeStruct((ns_rows, 128), jnp.float32),
        ),
        scratch_types=[
            pltpu.VMEM((npw,), jnp.int32),
            pltpu.VMEM((hpw,), jnp.int32),
            pltpu.VMEM((hpw,), jnp.int32),
            pltpu.VMEM((npw,), jnp.int32),
            pltpu.VMEM((npw, Dd), jnp.float32),
            pltpu.VMEM((npw, 128), jnp.float32),
            pltpu.SemaphoreType.DMA,
            pltpu.SemaphoreType.DMA,
            pltpu.SemaphoreType.DMA,
        ],
    )
    def dispatch(xf_hbm, tok_hbm, slot_hbm, p_hbm, xs_hbm, ps_hbm,
                 tok_v, slot_v0, slot_v1, slot_vf, xbuf, prep, sem_g, sem_x, sem_p):
        wid = lax.axis_index("s") * _NC + lax.axis_index("c")
        base = wid * npw
        pltpu.sync_copy(tok_hbm.at[pl.ds(base, npw)], tok_v)
        pltpu.sync_copy(slot_hbm.at[pl.ds(base, hpw)], slot_v0)
        pltpu.sync_copy(slot_hbm.at[pl.ds(base + hpw, hpw)], slot_v1)
        pltpu.sync_copy(slot_hbm.at[pl.ds(base, npw)], slot_vf)
        pltpu.sync_copy(p_hbm.at[pl.ds(base, npw)], prep)
        g0 = pltpu.async_copy(xf_hbm.at[tok_v.at[pl.ds(0, hpw)]],
                              xbuf.at[pl.ds(0, hpw)], sem_g)
        g1 = pltpu.async_copy(xf_hbm.at[tok_v.at[pl.ds(hpw, hpw)]],
                              xbuf.at[pl.ds(hpw, hpw)], sem_g)
        cp = pltpu.async_copy(prep, ps_hbm.at[slot_vf], sem_p)
        g0.wait()
        c0 = pltpu.async_copy(xbuf.at[pl.ds(0, hpw)], xs_hbm.at[slot_v0], sem_x)
        g1.wait()
        c1 = pltpu.async_copy(xbuf.at[pl.ds(hpw, hpw)], xs_hbm.at[slot_v1], sem_x)
        c0.wait()
        c1.wait()
        cp.wait()

    return dispatch


def _gelu_exact(v):
    return 0.5 * v * (1.0 + lax.erf(v * 0.7071067811865476))


def _mlp_body(bexp_ref, xs_ref, ps_ref, w1_ref, w2_ref, y_ref):
    h = jnp.dot(xs_ref[...].astype(jnp.bfloat16), w1_ref[...].astype(jnp.bfloat16),
                preferred_element_type=jnp.float32)
    h = _gelu_exact(h)
    y = jnp.dot(h.astype(jnp.bfloat16), w2_ref[...].astype(jnp.bfloat16),
                preferred_element_type=jnp.float32)
    y_ref[...] = y * ps_ref[:, 0:1]


def _grouped_mlp(bexp, xs, ps, w1, w2):
    ns_rows, Dd = xs.shape
    S = w1.shape[1] // E
    grid_spec = pltpu.PrefetchScalarGridSpec(
        num_scalar_prefetch=1,
        grid=(NBP,),
        in_specs=[
            pl.BlockSpec((BLK, Dd), lambda b, be: (b, 0)),
            pl.BlockSpec((BLK, 128), lambda b, be: (b, 0)),
            pl.BlockSpec((Dd, S), lambda b, be: (0, be[b])),
            pl.BlockSpec((S, Dd), lambda b, be: (be[b], 0)),
        ],
        out_specs=pl.BlockSpec((BLK, Dd), lambda b, be: (b, 0)),
    )
    return pl.pallas_call(
        _mlp_body,
        grid_spec=grid_spec,
        out_shape=jax.ShapeDtypeStruct((ns_rows, Dd), jnp.float32),
    )(bexp, xs, ps, w1, w2)


def _make_combine(T, Dd, ns_rows):
    npw = (T * K) // _NW
    tpw = T // _NW
    nch = Dd // _L
    mesh = plsc.VectorSubcoreMesh(core_axis_name="c", subcore_axis_name="s")

    @functools.partial(
        pl.kernel, mesh=mesh,
        out_type=jax.ShapeDtypeStruct((T * K, Dd), jnp.float32),
        scratch_types=[
            pltpu.VMEM((npw,), jnp.int32),
            pltpu.VMEM((npw, Dd), jnp.float32),
            pltpu.SemaphoreType.DMA,
        ],
    )
    def combine_gather(y_hbm, slot_hbm, yg_hbm, slot_v, ybuf, sem):
        wid = lax.axis_index("s") * _NC + lax.axis_index("c")
        base = wid * npw
        pltpu.sync_copy(slot_hbm.at[pl.ds(base, npw)], slot_v)
        pltpu.async_copy(y_hbm.at[slot_v], ybuf, sem).wait()
        pltpu.sync_copy(ybuf, yg_hbm.at[pl.ds(base, npw)])

    return combine_gather


def _pair_add_body(yg_ref, o_ref):
    R = o_ref.shape[0]
    ir = lax.broadcasted_iota(jnp.int32, (R, 2 * R), 0)
    ic = lax.broadcasted_iota(jnp.int32, (R, 2 * R), 1)
    pmat = ((ic == 2 * ir) | (ic == 2 * ir + 1)).astype(jnp.float32)
    o_ref[...] = jnp.dot(pmat, yg_ref[...], preferred_element_type=jnp.float32)


def _pair_add(yg, T, Dd):
    R = 512
    return pl.pallas_call(
        _pair_add_body,
        grid=(T // R,),
        in_specs=[pl.BlockSpec((2 * R, Dd), lambda w: (w, 0))],
        out_specs=pl.BlockSpec((R, Dd), lambda w: (w, 0)),
        out_shape=jax.ShapeDtypeStruct((T, Dd), jnp.float32),
    )(yg)


def kernel(x, w1, w2, router_w):
    Bb, Ss, Dd = x.shape
    T = Bb * Ss
    ns_rows = NBP * BLK
    xf = x.reshape(T, Dd)
    rwt = router_w.T

    slot, p, bexp = _route(xf, rwt)
    slot_flat = slot.reshape(T * K)
    p_rep = p.reshape(T * K, 128)
    bexp_flat = bexp.reshape(128)[:NBP]
    tok_ids = jnp.repeat(jnp.arange(T, dtype=jnp.int32), K)

    xs, ps = _make_dispatch(T, Dd, ns_rows)(xf, tok_ids, slot_flat, p_rep)
    y = _grouped_mlp(bexp_flat, xs, ps, w1, w2)
    yg = _make_combine(T, Dd, ns_rows)(y, slot_flat)
    out = _pair_add(yg, T, Dd)
    return out.reshape(Bb, Ss, Dd)


# speedup vs baseline: 1.6230x; 1.4336x over previous
import jax
import jax.numpy as jnp
from jax import lax
from jax.experimental import pallas as pl
from jax.experimental.pallas import tpu as pltpu

E = 8
K = 2
TB = 256


def _routing_weights(logits):
    T = logits.shape[0]
    iota_e = lax.broadcasted_iota(jnp.int32, (T, E), 1)
    m1 = jnp.max(logits, axis=1, keepdims=True)
    i1 = jnp.min(jnp.where(logits == m1, iota_e, E), axis=1, keepdims=True)
    l2 = jnp.where(iota_e == i1, -jnp.inf, logits)
    m2 = jnp.max(l2, axis=1, keepdims=True)
    i2 = jnp.min(jnp.where(l2 == m2, iota_e, E), axis=1, keepdims=True)
    p1 = 1.0 / (1.0 + jnp.exp(m2 - m1))
    p2 = 1.0 - p1
    return jnp.where(iota_e == i1, p1, 0.0) + jnp.where(iota_e == i2, p2, 0.0)


def _gelu_exact(v):
    return 0.5 * v * (1.0 + lax.erf(v * 0.7071067811865476))


def _moe_body(x_ref, rwt_ref, w1_ref, w2_ref, out_ref, wfull_scr):
    e = pl.program_id(0)
    i = pl.program_id(1)

    @pl.when((e == 0) & (i == 0))
    def _():
        logits = jnp.dot(x_ref[...], rwt_ref[...], preferred_element_type=jnp.float32)
        wfull_scr[...] = _routing_weights(logits)

    sl = pl.ds(i * TB, TB)
    h = jnp.dot(x_ref[sl, :].astype(jnp.bfloat16), w1_ref[...].astype(jnp.bfloat16),
                preferred_element_type=jnp.float32)
    h = _gelu_exact(h)
    y = jnp.dot(h.astype(jnp.bfloat16), w2_ref[...].astype(jnp.bfloat16),
                preferred_element_type=jnp.float32)
    esel = (lax.broadcasted_iota(jnp.int32, (1, E), 1) == e).astype(jnp.float32)
    wsel = jnp.sum(wfull_scr[sl, :] * esel, axis=1, keepdims=True)
    val = y * wsel

    @pl.when(e == 0)
    def _():
        out_ref[sl, :] = val

    @pl.when(e > 0)
    def _():
        out_ref[sl, :] += val


def kernel(x, w1, w2, router_w):
    Bb, Ss, Dd = x.shape
    T = Bb * Ss
    S = w1.shape[1] // E
    xf = x.reshape(T, Dd)
    rwt = router_w.T

    out = pl.pallas_call(
        _moe_body,
        grid=(E, T // TB),
        in_specs=[
            pl.BlockSpec((T, Dd), lambda e, i: (0, 0)),
            pl.BlockSpec((Dd, E), lambda e, i: (0, 0)),
            pl.BlockSpec((Dd, S), lambda e, i: (0, e)),
            pl.BlockSpec((S, Dd), lambda e, i: (e, 0)),
        ],
        out_specs=pl.BlockSpec((T, Dd), lambda e, i: (0, 0)),
        out_shape=jax.ShapeDtypeStruct((T, Dd), jnp.float32),
        scratch_shapes=[pltpu.VMEM((T, E), jnp.float32)],
    )(xf, rwt, w1, w2)
    return out.reshape(Bb, Ss, Dd)
